# baseline (speedup 1.0000x reference)
import functools

import jax
import jax.numpy as jnp
from jax import lax
from jax.experimental import pallas as pl
from jax.experimental.pallas import tpu as pltpu
from jax.experimental.pallas import tpu_sc as plsc

T, N, F, P, H, B = 8, 10000, 128, 16, 128, 8
E = 320000
NPART = N // B
KTOP = 1000
NPAD = 1280

NSUB = 16
NCORE = 2
LANES = 128
EP = 327680
ROWS = EP // LANES
RPS = ROWS // NSUB
CH = 8
NCHUNK = RPS // CH
NPSA = 624
TAIL = N - NSUB * NPSA
TPC = T // NCORE

@functools.cache
def _sc_params():
    import dataclasses
    cp = pltpu.CompilerParams(use_tc_tiling_on_sc=False)
    if "needs_layout_passes" in pltpu.CompilerParams.__dataclass_fields__:
        cp = dataclasses.replace(cp, needs_layout_passes=False)
    return cp


@functools.cache
def _mesh():
    return plsc.VectorSubcoreMesh(core_axis_name="core",
                                  subcore_axis_name="subcore",
                                  num_cores=NCORE, num_subcores=NSUB)
_HI = jax.lax.Precision.HIGHEST


def _sc_scratch():
    return [
        pltpu.VMEM_SHARED((N, P), jnp.float32),
        pltpu.VMEM((CH, LANES), jnp.int32),
        pltpu.VMEM((CH, LANES), jnp.int32),
        pltpu.VMEM((CH * LANES, P), jnp.float32),
        pltpu.VMEM((CH * LANES, P), jnp.float32),
        pltpu.SemaphoreType.DMA,
        pltpu.SemaphoreType.DMA,
        pltpu.SemaphoreType.DMA,
    ]


def _scale_rows(rows, wexp):
    @pl.loop(0, CH * LANES, unroll=8)
    def _m(e):
        rows[e] = rows[e] * wexp[e]


def _sc_body(gather, t, src_h, dst_h, w_h, y_h, z_h, out_h,
             accum, sidx, didx, wexp, rows, lsem, gsem, ssem, sid, nbase):
    pltpu.sync_copy(z_h, accum.at[pl.ds(nbase, NPSA)])

    @pl.when(sid == NSUB - 1)
    def _zt():
        pltpu.sync_copy(z_h.at[pl.ds(0, TAIL)],
                        accum.at[pl.ds(NSUB * NPSA, TAIL)])
    plsc.subcore_barrier()

    @pl.loop(0, NCHUNK)
    def _c(j):
        rb = sid * RPS + j * CH
        eb = rb * LANES
        hs = [pltpu.async_copy(w_h.at[t, pl.ds(eb, CH * LANES)], wexp, lsem),
              pltpu.async_copy(dst_h.at[t, pl.ds(rb, CH)], didx, lsem)]
        if gather:
            hs.append(pltpu.async_copy(src_h.at[t, pl.ds(rb, CH)], sidx, lsem))
        for h in hs:
            h.wait()
        if gather:
            gh = [pltpu.async_copy(y_h.at[t].at[sidx.at[r]],
                                   rows.at[pl.ds(r * LANES, LANES)], gsem)
                  for r in range(CH)]
            for h in gh:
                h.wait()
            _scale_rows(rows, wexp)
            srcbuf = rows
        else:
            srcbuf = wexp
        sh = [pltpu.async_copy(srcbuf.at[pl.ds(r * LANES, LANES)],
                               accum.at[didx.at[r]], ssem, add=True)
              for r in range(CH)]
        for h in sh:
            h.wait()

    plsc.subcore_barrier()
    pltpu.sync_copy(accum.at[pl.ds(nbase, NPSA)],
                    out_h.at[t, pl.ds(nbase, NPSA)])

    @pl.when(sid == NSUB - 1)
    def _ot():
        pltpu.sync_copy(accum.at[pl.ds(NSUB * NPSA, TAIL)],
                        out_h.at[t, pl.ds(NSUB * NPSA, TAIL)])


def _sc_gs(src3, dst3, w16, y, zeros):
    @functools.partial(
        pl.kernel,
        out_type=jax.ShapeDtypeStruct((T, N, P), jnp.float32),
        mesh=_mesh(),
        scratch_types=_sc_scratch(),
        compiler_params=_sc_params(),
    )
    def k(src_h, dst_h, w_h, y_h, z_h, out_h,
          accum, sidx, didx, wexp, rows, lsem, gsem, ssem):
        cid = lax.axis_index("core")
        sid = lax.axis_index("subcore")
        nbase = sid * NPSA

        @pl.loop(0, TPC)
        def _t(i):
            t = cid * TPC + i
            _sc_body(True, t, src_h, dst_h, w_h, y_h, z_h, out_h,
                     accum, sidx, didx, wexp, rows, lsem, gsem, ssem,
                     sid, nbase)

    return k(src3, dst3, w16, y, zeros)


def _sc_splat(dst3, w16, zeros):
    @functools.partial(
        pl.kernel,
        out_type=jax.ShapeDtypeStruct((T, N, P), jnp.float32),
        mesh=_mesh(),
        scratch_types=_sc_scratch(),
        compiler_params=_sc_params(),
    )
    def k(dst_h, w_h, z_h, out_h,
          accum, sidx, didx, wexp, rows, lsem, gsem, ssem):
        cid = lax.axis_index("core")
        sid = lax.axis_index("subcore")
        nbase = sid * NPSA

        @pl.loop(0, TPC)
        def _t(i):
            t = cid * TPC + i
            _sc_body(False, t, dst_h, dst_h, w_h, None, z_h, out_h,
                     accum, sidx, didx, wexp, rows, lsem, gsem, ssem,
                     sid, nbase)

    return k(dst3, w16, zeros)


BLKN = 2000


def _tc_xw(x, W1):
    def body(x_ref, w_ref, o_ref):
        o_ref[...] = lax.dot_general(
            x_ref[0], w_ref[...], (((1,), (0,)), ((), ())),
            preferred_element_type=jnp.float32, precision=_HI)[None]

    return pl.pallas_call(
        body,
        grid=(T, N // BLKN),
        in_specs=[pl.BlockSpec((1, BLKN, F), lambda t, i: (t, i, 0)),
                  pl.BlockSpec((F, P), lambda t, i: (0, 0))],
        out_specs=pl.BlockSpec((1, BLKN, P), lambda t, i: (t, i, 0)),
        out_shape=jax.ShapeDtypeStruct((T, N, P), jnp.float32),
    )(x, W1)


def _ew_specs(n_in):
    return dict(
        grid=(T, N // BLKN),
        in_specs=[pl.BlockSpec((1, BLKN, P), lambda t, i: (t, i, 0))
                  for _ in range(n_in)],
        out_specs=pl.BlockSpec((1, BLKN, P), lambda t, i: (t, i, 0)),
    )


def _tc_y1(D1, xw):
    def body(d_ref, x_ref, o_ref):
        d = lax.rsqrt(d_ref[...] + 1.0)
        o_ref[...] = d * x_ref[...]

    return pl.pallas_call(
        body, **_ew_specs(2),
        out_shape=jax.ShapeDtypeStruct((T, N, P), jnp.float32))(D1, xw)


def _tc_h1(D1, xw, A1, b1r):
    def body(d_ref, x_ref, a_ref, b_ref, o_ref):
        d = lax.rsqrt(d_ref[...] + 1.0)
        o_ref[...] = jnp.maximum(
            d * a_ref[...] + d * d * x_ref[...] + b_ref[...], 0.0)

    sp = _ew_specs(3)
    sp["in_specs"].append(pl.BlockSpec((1, 1, P), lambda t, i: (0, 0, 0)))
    return pl.pallas_call(
        body, **sp,
        out_shape=jax.ShapeDtypeStruct((T, N, P), jnp.float32))(D1, xw, A1, b1r)


def _tc_score(AG, h1, Wrel_r, Wroot_r, brel_r):
    def body(ag_ref, h_ref, wr_ref, wo_ref, br_ref, o_ref):
        s = (jnp.sum(ag_ref[...] * wr_ref[...], axis=2, keepdims=True)
             + jnp.sum(h_ref[...] * wo_ref[...], axis=2, keepdims=True)
             + br_ref[...])
        o_ref[...] = jnp.broadcast_to(s, (1, BLKN, P))

    sp = _ew_specs(2)
    sp["in_specs"] += [pl.BlockSpec((1, 1, P), lambda t, i: (0, 0, 0)),
                       pl.BlockSpec((1, 1, P), lambda t, i: (0, 0, 0)),
                       pl.BlockSpec((1, 1, 1), lambda t, i: (0, 0, 0))]
    return pl.pallas_call(
        body, **sp,
        out_shape=jax.ShapeDtypeStruct((T, N, P), jnp.float32))(
            AG, h1, Wrel_r, Wroot_r, brel_r)


def _tc_keep(sp_row, sp_col):
    def body(r_ref, c_ref, o_ref):
        srow = r_ref[0]
        scol = c_ref[0]
        ii = lax.broadcasted_iota(jnp.int32, (NPAD, NPAD), 0)
        jj = lax.broadcasted_iota(jnp.int32, (NPAD, NPAD), 1)
        gt = (srow > scol).astype(jnp.float32)
        eqb = ((srow == scol) & (jj < ii)).astype(jnp.float32)
        cnt = jnp.sum(gt + eqb, axis=1, keepdims=True)
        keep = (cnt < float(KTOP)).astype(jnp.float32)
        o_ref[...] = jnp.broadcast_to(keep, (NPAD, P))[None]

    return pl.pallas_call(
        body,
        grid=(T * B,),
        in_specs=[pl.BlockSpec((1, 1, NPAD), lambda g: (g, 0, 0)),
                  pl.BlockSpec((1, NPAD, 1), lambda g: (g, 0, 0))],
        out_specs=pl.BlockSpec((1, NPAD, P), lambda g: (g, 0, 0)),
        out_shape=jax.ShapeDtypeStruct((T * B, NPAD, P), jnp.float32),
    )(sp_row, sp_col)


def _tc_y2xk(D2, keepw, h1, scw):
    def body(d_ref, k_ref, h_ref, s_ref, y_ref, xk_ref, dv_ref):
        d2 = k_ref[...] * lax.rsqrt(d_ref[...] + 1.0)
        xk = k_ref[...] * h_ref[...] * jnp.tanh(s_ref[...])
        xk_ref[...] = xk
        dv_ref[...] = d2
        y_ref[...] = d2 * xk

    sh = jax.ShapeDtypeStruct((T, N, P), jnp.float32)
    sp = _ew_specs(4)
    sp["out_specs"] = (sp["out_specs"],) * 3
    return pl.pallas_call(
        body, **sp, out_shape=(sh, sh, sh))(D2, keepw, h1, scw)


def _tc_emb(d2w, xk, A2, keepw, W2, b2r, sel):
    def body(d_ref, xk_ref, a_ref, k_ref, w_ref, b_ref, s_ref, o_ref):
        d2 = d_ref[0]
        pre = d2 * a_ref[0] + d2 * d2 * xk_ref[0]
        g = jnp.maximum(
            lax.dot_general(pre, w_ref[...], (((1,), (0,)), ((), ())),
                            preferred_element_type=jnp.float32,
                            precision=_HI) + b_ref[...], 0.0)
        g = g * k_ref[0][:, 0:1]
        o_ref[...] = lax.dot_general(
            s_ref[...], g, (((1,), (0,)), ((), ())),
            preferred_element_type=jnp.float32, precision=_HI)[None]

    return pl.pallas_call(
        body,
        grid=(T,),
        in_specs=[pl.BlockSpec((1, N, P), lambda t: (t, 0, 0)),
                  pl.BlockSpec((1, N, P), lambda t: (t, 0, 0)),
                  pl.BlockSpec((1, N, P), lambda t: (t, 0, 0)),
                  pl.BlockSpec((1, N, P), lambda t: (t, 0, 0)),
                  pl.BlockSpec((P, H), lambda t: (0, 0)),
                  pl.BlockSpec((1, H), lambda t: (0, 0)),
                  pl.BlockSpec((B, N), lambda t: (0, 0))],
        out_specs=pl.BlockSpec((1, B, H), lambda t: (t, 0, 0)),
        out_shape=jax.ShapeDtypeStruct((T, B, H), jnp.float32),
    )(d2w, xk, A2, keepw, W2, b2r, sel)


def _tc_lstm(emb, Wih, Whh, bihr, bhhr, Wo, bor):
    def body(s_ref, wi_ref, wh_ref, bi_ref, bh_ref, wo_ref, bo_ref, o_ref):
        h = jnp.zeros((B, H), jnp.float32)
        c = jnp.zeros((B, H), jnp.float32)
        for t in range(T):
            st = s_ref[t]
            gates = (lax.dot_general(st, wi_ref[...], (((1,), (1,)), ((), ())),
                                     preferred_element_type=jnp.float32,
                                     precision=_HI)
                     + lax.dot_general(h, wh_ref[...], (((1,), (1,)), ((), ())),
                                       preferred_element_type=jnp.float32,
                                       precision=_HI)
                     + bi_ref[...] + bh_ref[...])
            ig = jax.nn.sigmoid(gates[:, 0:H])
            fg = jax.nn.sigmoid(gates[:, H:2 * H])
            gg = jnp.tanh(gates[:, 2 * H:3 * H])
            og = jax.nn.sigmoid(gates[:, 3 * H:4 * H])
            c = fg * c + ig * gg
            h = og * jnp.tanh(c)
        o_ref[...] = jax.nn.sigmoid(
            lax.dot_general(h, wo_ref[...], (((1,), (0,)), ((), ())),
                            preferred_element_type=jnp.float32,
                            precision=_HI) + bo_ref[...])

    return pl.pallas_call(
        body,
        in_specs=[pl.BlockSpec((T, B, H), lambda: (0, 0, 0)),
                  pl.BlockSpec((4 * H, H), lambda: (0, 0)),
                  pl.BlockSpec((4 * H, H), lambda: (0, 0)),
                  pl.BlockSpec((1, 4 * H), lambda: (0, 0)),
                  pl.BlockSpec((1, 4 * H), lambda: (0, 0)),
                  pl.BlockSpec((H, 1), lambda: (0, 0)),
                  pl.BlockSpec((1, 1), lambda: (0, 0))],
        out_specs=pl.BlockSpec((B, 1), lambda: (0, 0)),
        out_shape=jax.ShapeDtypeStruct((B, 1), jnp.float32),
    )(emb, Wih, Whh, bihr, bhhr, Wo, bor)


def kernel(x, edge_index, edge_weight, W1, b1, Wrel, brel, Wroot, W2, b2,
           Wih, Whh, bih, bhh, Wo, bo):
    src = edge_index[:, 0, :].astype(jnp.int32)
    dst = edge_index[:, 1, :].astype(jnp.int32)
    padi = jnp.zeros((T, EP - E), jnp.int32)
    padf = jnp.zeros((T, EP - E), jnp.float32)
    src3 = jnp.concatenate([src, padi], axis=1).reshape(T, ROWS, LANES)
    dst3 = jnp.concatenate([dst, padi], axis=1).reshape(T, ROWS, LANES)
    wp = jnp.concatenate([edge_weight, padf], axis=1)
    w16 = jnp.broadcast_to(wp[:, :, None], (T, EP, P))
    zeros = jnp.zeros((NPSA, P), jnp.float32)
    sel = jnp.repeat(jnp.eye(B, dtype=jnp.float32), NPART, axis=1) / KTOP

    xw = _tc_xw(x, W1)
    D1 = _sc_splat(dst3, w16, zeros)
    y1 = _tc_y1(D1, xw)
    A1 = _sc_gs(src3, dst3, w16, y1, zeros)
    h1 = _tc_h1(D1, xw, A1, b1.reshape(1, 1, P))
    AG = _sc_gs(src3, dst3, w16, h1, zeros)
    scw = _tc_score(AG, h1, Wrel.reshape(1, 1, P), Wroot.reshape(1, 1, P),
                    brel.reshape(1, 1, 1))
    sc_nodes = scw[:, :, 0].reshape(T * B, NPART)
    sp = jnp.pad(sc_nodes, ((0, 0), (0, NPAD - NPART)), constant_values=-1e30)
    keep64 = _tc_keep(sp.reshape(T * B, 1, NPAD), sp.reshape(T * B, NPAD, 1))
    keepw = keep64[:, :NPART, :].reshape(T, N, P)
    D2 = _sc_gs(src3, dst3, w16, keepw, zeros)
    y2, xk, d2w = _tc_y2xk(D2, keepw, h1, scw)
    A2 = _sc_gs(src3, dst3, w16, y2, zeros)
    emb = _tc_emb(d2w, xk, A2, keepw, W2, b2.reshape(1, H), sel)
    return _tc_lstm(emb, Wih, Whh, bih.reshape(1, 4 * H), bhh.reshape(1, 4 * H),
                    Wo, bo.reshape(1, 1))


# speedup vs baseline: 1.4077x; 1.4077x over previous
import functools

import jax
import jax.numpy as jnp
from jax import lax
from jax.experimental import pallas as pl
from jax.experimental.pallas import tpu as pltpu
from jax.experimental.pallas import tpu_sc as plsc

T, N, F, P, H, B = 8, 10000, 128, 16, 128, 8
E = 320000
NPART = N // B
KTOP = 1000
NPAD = 1280

NSUB = 16
NCORE = 2
LANES = 128
EP = 327680
ROWS = EP // LANES
RPS = ROWS // NSUB
CH = 8
NCHUNK = RPS // CH
NPSA = 624
TAIL = N - NSUB * NPSA
TPC = T // NCORE

@functools.cache
def _sc_params():
    import dataclasses
    cp = pltpu.CompilerParams(use_tc_tiling_on_sc=False)
    if "needs_layout_passes" in pltpu.CompilerParams.__dataclass_fields__:
        cp = dataclasses.replace(cp, needs_layout_passes=False)
    return cp


@functools.cache
def _mesh():
    return plsc.VectorSubcoreMesh(core_axis_name="core",
                                  subcore_axis_name="subcore",
                                  num_cores=NCORE, num_subcores=NSUB)
_HI = jax.lax.Precision.HIGHEST


def _sc_scratch():
    buf = []
    for _ in range(2):
        buf += [
            pltpu.VMEM((CH, LANES), jnp.int32),
            pltpu.VMEM((CH, LANES), jnp.int32),
            pltpu.VMEM((CH, LANES), jnp.float32),
            pltpu.VMEM((CH * LANES, P), jnp.float32),
            pltpu.SemaphoreType.DMA,
            pltpu.SemaphoreType.DMA,
            pltpu.SemaphoreType.DMA,
        ]
    return [pltpu.VMEM_SHARED((N, P), jnp.float32)] + buf


def _scale_rows(gather, rows, wbuf):
    @pl.loop(0, CH)
    def _mr(r):
        rsp = jnp.zeros((P,), jnp.int32) + r

        @pl.loop(0, LANES, unroll=4)
        def _ml(l):
            lsp = jnp.zeros((P,), jnp.int32) + l
            wsp = plsc.load_gather(wbuf, [rsp, lsp])
            e = r * LANES + l
            if gather:
                rows[e] = rows[e] * wsp
            else:
                rows[e] = wsp


def _sc_body(gather, t, src_h, dst_h, w_h, y_h, z_h, out_h,
             accum, b0, b1, sid, nbase):
    pltpu.sync_copy(z_h, accum.at[pl.ds(nbase, NPSA)])

    @pl.when(sid == NSUB - 1)
    def _zt():
        pltpu.sync_copy(z_h.at[pl.ds(0, TAIL)],
                        accum.at[pl.ds(NSUB * NPSA, TAIL)])
    plsc.subcore_barrier()

    def fire_lin(j, bufs):
        sidx, didx, wbuf, rows, lsem, gsem, ssem = bufs
        rb = sid * RPS + j * CH
        hs = [pltpu.async_copy(dst_h.at[t, pl.ds(rb, CH)], didx, lsem),
              pltpu.async_copy(w_h.at[t, pl.ds(rb, CH)], wbuf, lsem)]
        if gather:
            hs.append(pltpu.async_copy(src_h.at[t, pl.ds(rb, CH)], sidx, lsem))
        return hs

    def fire_gather(bufs):
        sidx, didx, wbuf, rows, lsem, gsem, ssem = bufs
        if not gather:
            return []
        return [pltpu.async_copy(y_h.at[t].at[sidx.at[r]],
                                 rows.at[pl.ds(r * LANES, LANES)], gsem)
                for r in range(CH)]

    def fire_scatter(bufs):
        sidx, didx, wbuf, rows, lsem, gsem, ssem = bufs
        return [pltpu.async_copy(rows.at[pl.ds(r * LANES, LANES)],
                                 accum.at[didx.at[r]], ssem, add=True)
                for r in range(CH)]

    @pl.loop(0, NCHUNK // 2)
    def _c(k):
        a, b = 2 * k, 2 * k + 1
        la = fire_lin(a, b0)
        lb = fire_lin(b, b1)
        for h in la:
            h.wait()
        ga = fire_gather(b0)
        for h in lb:
            h.wait()
        for h in ga:
            h.wait()
        gb = fire_gather(b1)
        _scale_rows(gather, b0[3], b0[2])
        sa = fire_scatter(b0)
        for h in gb:
            h.wait()
        _scale_rows(gather, b1[3], b1[2])
        for h in sa:
            h.wait()
        sb = fire_scatter(b1)
        for h in sb:
            h.wait()

    plsc.subcore_barrier()
    pltpu.sync_copy(accum.at[pl.ds(nbase, NPSA)],
                    out_h.at[t, pl.ds(nbase, NPSA)])

    @pl.when(sid == NSUB - 1)
    def _ot():
        pltpu.sync_copy(accum.at[pl.ds(NSUB * NPSA, TAIL)],
                        out_h.at[t, pl.ds(NSUB * NPSA, TAIL)])


def _sc_gs(src3, dst3, w3, y, zeros):
    @functools.partial(
        pl.kernel,
        out_type=jax.ShapeDtypeStruct((T, N, P), jnp.float32),
        mesh=_mesh(),
        scratch_types=_sc_scratch(),
        compiler_params=_sc_params(),
    )
    def k(src_h, dst_h, w_h, y_h, z_h, out_h, accum, *bufs):
        cid = lax.axis_index("core")
        sid = lax.axis_index("subcore")
        nbase = sid * NPSA

        @pl.loop(0, TPC)
        def _t(i):
            t = cid * TPC + i
            _sc_body(True, t, src_h, dst_h, w_h, y_h, z_h, out_h,
                     accum, bufs[:7], bufs[7:], sid, nbase)

    return k(src3, dst3, w3, y, zeros)


def _sc_splat(dst3, w3, zeros):
    @functools.partial(
        pl.kernel,
        out_type=jax.ShapeDtypeStruct((T, N, P), jnp.float32),
        mesh=_mesh(),
        scratch_types=_sc_scratch(),
        compiler_params=_sc_params(),
    )
    def k(dst_h, w_h, z_h, out_h, accum, *bufs):
        cid = lax.axis_index("core")
        sid = lax.axis_index("subcore")
        nbase = sid * NPSA

        @pl.loop(0, TPC)
        def _t(i):
            t = cid * TPC + i
            _sc_body(False, t, dst_h, dst_h, w_h, None, z_h, out_h,
                     accum, bufs[:7], bufs[7:], sid, nbase)

    return k(dst3, w3, zeros)


BLKN = 2000


def _tc_xw(x, W1):
    def body(x_ref, w_ref, o_ref):
        o_ref[...] = lax.dot_general(
            x_ref[0], w_ref[...], (((1,), (0,)), ((), ())),
            preferred_element_type=jnp.float32, precision=_HI)[None]

    return pl.pallas_call(
        body,
        grid=(T, N // BLKN),
        in_specs=[pl.BlockSpec((1, BLKN, F), lambda t, i: (t, i, 0)),
                  pl.BlockSpec((F, P), lambda t, i: (0, 0))],
        out_specs=pl.BlockSpec((1, BLKN, P), lambda t, i: (t, i, 0)),
        out_shape=jax.ShapeDtypeStruct((T, N, P), jnp.float32),
    )(x, W1)


def _ew_specs(n_in):
    return dict(
        grid=(T, N // BLKN),
        in_specs=[pl.BlockSpec((1, BLKN, P), lambda t, i: (t, i, 0))
                  for _ in range(n_in)],
        out_specs=pl.BlockSpec((1, BLKN, P), lambda t, i: (t, i, 0)),
    )


def _tc_y1(D1, xw):
    def body(d_ref, x_ref, o_ref):
        d = lax.rsqrt(d_ref[...] + 1.0)
        o_ref[...] = d * x_ref[...]

    return pl.pallas_call(
        body, **_ew_specs(2),
        out_shape=jax.ShapeDtypeStruct((T, N, P), jnp.float32))(D1, xw)


def _tc_h1(D1, xw, A1, b1r):
    def body(d_ref, x_ref, a_ref, b_ref, o_ref):
        d = lax.rsqrt(d_ref[...] + 1.0)
        o_ref[...] = jnp.maximum(
            d * a_ref[...] + d * d * x_ref[...] + b_ref[...], 0.0)

    sp = _ew_specs(3)
    sp["in_specs"].append(pl.BlockSpec((1, 1, P), lambda t, i: (0, 0, 0)))
    return pl.pallas_call(
        body, **sp,
        out_shape=jax.ShapeDtypeStruct((T, N, P), jnp.float32))(D1, xw, A1, b1r)


def _tc_score(AG, h1, Wrel_r, Wroot_r, brel_r):
    def body(ag_ref, h_ref, wr_ref, wo_ref, br_ref, o_ref):
        s = (jnp.sum(ag_ref[...] * wr_ref[...], axis=2, keepdims=True)
             + jnp.sum(h_ref[...] * wo_ref[...], axis=2, keepdims=True)
             + br_ref[...])
        o_ref[...] = jnp.broadcast_to(s, (1, BLKN, P))

    sp = _ew_specs(2)
    sp["in_specs"] += [pl.BlockSpec((1, 1, P), lambda t, i: (0, 0, 0)),
                       pl.BlockSpec((1, 1, P), lambda t, i: (0, 0, 0)),
                       pl.BlockSpec((1, 1, 1), lambda t, i: (0, 0, 0))]
    return pl.pallas_call(
        body, **sp,
        out_shape=jax.ShapeDtypeStruct((T, N, P), jnp.float32))(
            AG, h1, Wrel_r, Wroot_r, brel_r)


def _tc_keep(sp_row, sp_col):
    def body(r_ref, c_ref, o_ref):
        srow = r_ref[0]
        scol = c_ref[0]
        ii = lax.broadcasted_iota(jnp.int32, (NPAD, NPAD), 0)
        jj = lax.broadcasted_iota(jnp.int32, (NPAD, NPAD), 1)
        gt = (srow > scol).astype(jnp.float32)
        eqb = ((srow == scol) & (jj < ii)).astype(jnp.float32)
        cnt = jnp.sum(gt + eqb, axis=1, keepdims=True)
        keep = (cnt < float(KTOP)).astype(jnp.float32)
        o_ref[...] = jnp.broadcast_to(keep, (NPAD, P))[None]

    return pl.pallas_call(
        body,
        grid=(T * B,),
        in_specs=[pl.BlockSpec((1, 1, NPAD), lambda g: (g, 0, 0)),
                  pl.BlockSpec((1, NPAD, 1), lambda g: (g, 0, 0))],
        out_specs=pl.BlockSpec((1, NPAD, P), lambda g: (g, 0, 0)),
        out_shape=jax.ShapeDtypeStruct((T * B, NPAD, P), jnp.float32),
    )(sp_row, sp_col)


def _tc_y2xk(D2, keepw, h1, scw):
    def body(d_ref, k_ref, h_ref, s_ref, y_ref, xk_ref, dv_ref):
        d2 = k_ref[...] * lax.rsqrt(d_ref[...] + 1.0)
        xk = k_ref[...] * h_ref[...] * jnp.tanh(s_ref[...])
        xk_ref[...] = xk
        dv_ref[...] = d2
        y_ref[...] = d2 * xk

    sh = jax.ShapeDtypeStruct((T, N, P), jnp.float32)
    sp = _ew_specs(4)
    sp["out_specs"] = (sp["out_specs"],) * 3
    return pl.pallas_call(
        body, **sp, out_shape=(sh, sh, sh))(D2, keepw, h1, scw)


def _tc_emb(d2w, xk, A2, keepw, W2, b2r, sel):
    def body(d_ref, xk_ref, a_ref, k_ref, w_ref, b_ref, s_ref, o_ref):
        d2 = d_ref[0]
        pre = d2 * a_ref[0] + d2 * d2 * xk_ref[0]
        g = jnp.maximum(
            lax.dot_general(pre, w_ref[...], (((1,), (0,)), ((), ())),
                            preferred_element_type=jnp.float32,
                            precision=_HI) + b_ref[...], 0.0)
        g = g * k_ref[0][:, 0:1]
        o_ref[...] = lax.dot_general(
            s_ref[...], g, (((1,), (0,)), ((), ())),
            preferred_element_type=jnp.float32, precision=_HI)[None]

    return pl.pallas_call(
        body,
        grid=(T,),
        in_specs=[pl.BlockSpec((1, N, P), lambda t: (t, 0, 0)),
                  pl.BlockSpec((1, N, P), lambda t: (t, 0, 0)),
                  pl.BlockSpec((1, N, P), lambda t: (t, 0, 0)),
                  pl.BlockSpec((1, N, P), lambda t: (t, 0, 0)),
                  pl.BlockSpec((P, H), lambda t: (0, 0)),
                  pl.BlockSpec((1, H), lambda t: (0, 0)),
                  pl.BlockSpec((B, N), lambda t: (0, 0))],
        out_specs=pl.BlockSpec((1, B, H), lambda t: (t, 0, 0)),
        out_shape=jax.ShapeDtypeStruct((T, B, H), jnp.float32),
    )(d2w, xk, A2, keepw, W2, b2r, sel)


def _tc_lstm(emb, Wih, Whh, bihr, bhhr, Wo, bor):
    def body(s_ref, wi_ref, wh_ref, bi_ref, bh_ref, wo_ref, bo_ref, o_ref):
        h = jnp.zeros((B, H), jnp.float32)
        c = jnp.zeros((B, H), jnp.float32)
        for t in range(T):
            st = s_ref[t]
            gates = (lax.dot_general(st, wi_ref[...], (((1,), (1,)), ((), ())),
                                     preferred_element_type=jnp.float32,
                                     precision=_HI)
                     + lax.dot_general(h, wh_ref[...], (((1,), (1,)), ((), ())),
                                       preferred_element_type=jnp.float32,
                                       precision=_HI)
                     + bi_ref[...] + bh_ref[...])
            ig = jax.nn.sigmoid(gates[:, 0:H])
            fg = jax.nn.sigmoid(gates[:, H:2 * H])
            gg = jnp.tanh(gates[:, 2 * H:3 * H])
            og = jax.nn.sigmoid(gates[:, 3 * H:4 * H])
            c = fg * c + ig * gg
            h = og * jnp.tanh(c)
        o_ref[...] = jax.nn.sigmoid(
            lax.dot_general(h, wo_ref[...], (((1,), (0,)), ((), ())),
                            preferred_element_type=jnp.float32,
                            precision=_HI) + bo_ref[...])

    return pl.pallas_call(
        body,
        in_specs=[pl.BlockSpec((T, B, H), lambda: (0, 0, 0)),
                  pl.BlockSpec((4 * H, H), lambda: (0, 0)),
                  pl.BlockSpec((4 * H, H), lambda: (0, 0)),
                  pl.BlockSpec((1, 4 * H), lambda: (0, 0)),
                  pl.BlockSpec((1, 4 * H), lambda: (0, 0)),
                  pl.BlockSpec((H, 1), lambda: (0, 0)),
                  pl.BlockSpec((1, 1), lambda: (0, 0))],
        out_specs=pl.BlockSpec((B, 1), lambda: (0, 0)),
        out_shape=jax.ShapeDtypeStruct((B, 1), jnp.float32),
    )(emb, Wih, Whh, bihr, bhhr, Wo, bor)


def kernel(x, edge_index, edge_weight, W1, b1, Wrel, brel, Wroot, W2, b2,
           Wih, Whh, bih, bhh, Wo, bo):
    src = edge_index[:, 0, :].astype(jnp.int32)
    dst = edge_index[:, 1, :].astype(jnp.int32)
    padi = jnp.zeros((T, EP - E), jnp.int32)
    padf = jnp.zeros((T, EP - E), jnp.float32)
    src3 = jnp.concatenate([src, padi], axis=1).reshape(T, ROWS, LANES)
    dst3 = jnp.concatenate([dst, padi], axis=1).reshape(T, ROWS, LANES)
    w3 = jnp.concatenate([edge_weight, padf], axis=1).reshape(T, ROWS, LANES)
    zeros = jnp.zeros((NPSA, P), jnp.float32)
    sel = jnp.repeat(jnp.eye(B, dtype=jnp.float32), NPART, axis=1) / KTOP

    xw = _tc_xw(x, W1)
    D1 = _sc_splat(dst3, w3, zeros)
    y1 = _tc_y1(D1, xw)
    A1 = _sc_gs(src3, dst3, w3, y1, zeros)
    h1 = _tc_h1(D1, xw, A1, b1.reshape(1, 1, P))
    AG = _sc_gs(src3, dst3, w3, h1, zeros)
    scw = _tc_score(AG, h1, Wrel.reshape(1, 1, P), Wroot.reshape(1, 1, P),
                    brel.reshape(1, 1, 1))
    sc_nodes = scw[:, :, 0].reshape(T * B, NPART)
    sp = jnp.pad(sc_nodes, ((0, 0), (0, NPAD - NPART)), constant_values=-1e30)
    keep64 = _tc_keep(sp.reshape(T * B, 1, NPAD), sp.reshape(T * B, NPAD, 1))
    keepw = keep64[:, :NPART, :].reshape(T, N, P)
    D2 = _sc_gs(src3, dst3, w3, keepw, zeros)
    y2, xk, d2w = _tc_y2xk(D2, keepw, h1, scw)
    A2 = _sc_gs(src3, dst3, w3, y2, zeros)
    emb = _tc_emb(d2w, xk, A2, keepw, W2, b2.reshape(1, H), sel)
    return _tc_lstm(emb, Wih, Whh, bih.reshape(1, 4 * H), bhh.reshape(1, 4 * H),
                    Wo, bo.reshape(1, 1))


# speedup vs baseline: 1.4395x; 1.0226x over previous
import functools

import jax
import jax.numpy as jnp
from jax import lax
from jax.experimental import pallas as pl
from jax.experimental.pallas import tpu as pltpu
from jax.experimental.pallas import tpu_sc as plsc

T, N, F, P, H, B = 8, 10000, 128, 16, 128, 8
E = 320000
NPART = N // B
KTOP = 1000
NPAD = 1280

NSUB = 16
NCORE = 2
LANES = 128
EP = 327680
ROWS = EP // LANES
RPS = ROWS // NSUB
CH = 16
NCHUNK = RPS // CH
NPSA = 624
TAIL = N - NSUB * NPSA
TPC = T // NCORE

@functools.cache
def _sc_params():
    import dataclasses
    cp = pltpu.CompilerParams(use_tc_tiling_on_sc=False)
    if "needs_layout_passes" in pltpu.CompilerParams.__dataclass_fields__:
        cp = dataclasses.replace(cp, needs_layout_passes=False)
    return cp


@functools.cache
def _mesh():
    return plsc.VectorSubcoreMesh(core_axis_name="core",
                                  subcore_axis_name="subcore",
                                  num_cores=NCORE, num_subcores=NSUB)
_HI = jax.lax.Precision.HIGHEST


def _sc_scratch():
    buf = []
    for _ in range(2):
        buf += [
            pltpu.VMEM((CH, LANES), jnp.int32),
            pltpu.VMEM((CH, LANES), jnp.int32),
            pltpu.VMEM((CH, LANES), jnp.float32),
            pltpu.VMEM((CH * LANES, P), jnp.float32),
            pltpu.SemaphoreType.DMA,
            pltpu.SemaphoreType.DMA,
            pltpu.SemaphoreType.DMA,
        ]
    return [pltpu.VMEM_SHARED((N, P), jnp.float32)] + buf


def _scale_rows(gather, rows, wbuf):
    @pl.loop(0, CH)
    def _mr(r):
        rsp = jnp.zeros((P,), jnp.int32) + r

        @pl.loop(0, LANES, unroll=4)
        def _ml(l):
            lsp = jnp.zeros((P,), jnp.int32) + l
            wsp = plsc.load_gather(wbuf, [rsp, lsp])
            e = r * LANES + l
            if gather:
                rows[e] = rows[e] * wsp
            else:
                rows[e] = wsp


def _sc_body(gather, t, src_h, dst_h, w_h, y_h, z_h, out_h,
             accum, b0, b1, sid, nbase):
    pltpu.sync_copy(z_h, accum.at[pl.ds(nbase, NPSA)])

    @pl.when(sid == NSUB - 1)
    def _zt():
        pltpu.sync_copy(z_h.at[pl.ds(0, TAIL)],
                        accum.at[pl.ds(NSUB * NPSA, TAIL)])
    plsc.subcore_barrier()

    def fire_lin(j, bufs):
        sidx, didx, wbuf, rows, lsem, gsem, ssem = bufs
        rb = sid * RPS + j * CH
        hs = [pltpu.async_copy(dst_h.at[t, pl.ds(rb, CH)], didx, lsem),
              pltpu.async_copy(w_h.at[t, pl.ds(rb, CH)], wbuf, lsem)]
        if gather:
            hs.append(pltpu.async_copy(src_h.at[t, pl.ds(rb, CH)], sidx, lsem))
        return hs

    def fire_gather(bufs):
        sidx, didx, wbuf, rows, lsem, gsem, ssem = bufs
        if not gather:
            return []
        return [pltpu.async_copy(y_h.at[t].at[sidx.at[r]],
                                 rows.at[pl.ds(r * LANES, LANES)], gsem)
                for r in range(CH)]

    def fire_scatter(bufs):
        sidx, didx, wbuf, rows, lsem, gsem, ssem = bufs
        return [pltpu.async_copy(rows.at[pl.ds(r * LANES, LANES)],
                                 accum.at[didx.at[r]], ssem, add=True)
                for r in range(CH)]

    @pl.loop(0, NCHUNK // 2)
    def _c(k):
        a, b = 2 * k, 2 * k + 1
        la = fire_lin(a, b0)
        lb = fire_lin(b, b1)
        for h in la:
            h.wait()
        ga = fire_gather(b0)
        for h in lb:
            h.wait()
        for h in ga:
            h.wait()
        gb = fire_gather(b1)
        _scale_rows(gather, b0[3], b0[2])
        sa = fire_scatter(b0)
        for h in gb:
            h.wait()
        _scale_rows(gather, b1[3], b1[2])
        for h in sa:
            h.wait()
        sb = fire_scatter(b1)
        for h in sb:
            h.wait()

    plsc.subcore_barrier()
    pltpu.sync_copy(accum.at[pl.ds(nbase, NPSA)],
                    out_h.at[t, pl.ds(nbase, NPSA)])

    @pl.when(sid == NSUB - 1)
    def _ot():
        pltpu.sync_copy(accum.at[pl.ds(NSUB * NPSA, TAIL)],
                        out_h.at[t, pl.ds(NSUB * NPSA, TAIL)])


def _sc_gs(src3, dst3, w3, y, zeros):
    @functools.partial(
        pl.kernel,
        out_type=jax.ShapeDtypeStruct((T, N, P), jnp.float32),
        mesh=_mesh(),
        scratch_types=_sc_scratch(),
        compiler_params=_sc_params(),
    )
    def k(src_h, dst_h, w_h, y_h, z_h, out_h, accum, *bufs):
        cid = lax.axis_index("core")
        sid = lax.axis_index("subcore")
        nbase = sid * NPSA

        @pl.loop(0, TPC)
        def _t(i):
            t = cid * TPC + i
            _sc_body(True, t, src_h, dst_h, w_h, y_h, z_h, out_h,
                     accum, bufs[:7], bufs[7:], sid, nbase)

    return k(src3, dst3, w3, y, zeros)


def _sc_splat(dst3, w3, zeros):
    @functools.partial(
        pl.kernel,
        out_type=jax.ShapeDtypeStruct((T, N, P), jnp.float32),
        mesh=_mesh(),
        scratch_types=_sc_scratch(),
        compiler_params=_sc_params(),
    )
    def k(dst_h, w_h, z_h, out_h, accum, *bufs):
        cid = lax.axis_index("core")
        sid = lax.axis_index("subcore")
        nbase = sid * NPSA

        @pl.loop(0, TPC)
        def _t(i):
            t = cid * TPC + i
            _sc_body(False, t, dst_h, dst_h, w_h, None, z_h, out_h,
                     accum, bufs[:7], bufs[7:], sid, nbase)

    return k(dst3, w3, zeros)


BLKN = 2000


def _tc_xw(x, W1):
    def body(x_ref, w_ref, o_ref):
        o_ref[...] = lax.dot_general(
            x_ref[0], w_ref[...], (((1,), (0,)), ((), ())),
            preferred_element_type=jnp.float32, precision=_HI)[None]

    return pl.pallas_call(
        body,
        grid=(T, N // BLKN),
        in_specs=[pl.BlockSpec((1, BLKN, F), lambda t, i: (t, i, 0)),
                  pl.BlockSpec((F, P), lambda t, i: (0, 0))],
        out_specs=pl.BlockSpec((1, BLKN, P), lambda t, i: (t, i, 0)),
        out_shape=jax.ShapeDtypeStruct((T, N, P), jnp.float32),
    )(x, W1)


def _ew_specs(n_in):
    return dict(
        grid=(T, N // BLKN),
        in_specs=[pl.BlockSpec((1, BLKN, P), lambda t, i: (t, i, 0))
                  for _ in range(n_in)],
        out_specs=pl.BlockSpec((1, BLKN, P), lambda t, i: (t, i, 0)),
    )


def _tc_y1(D1, xw):
    def body(d_ref, x_ref, o_ref):
        d = lax.rsqrt(d_ref[...] + 1.0)
        o_ref[...] = d * x_ref[...]

    return pl.pallas_call(
        body, **_ew_specs(2),
        out_shape=jax.ShapeDtypeStruct((T, N, P), jnp.float32))(D1, xw)


def _tc_h1(D1, xw, A1, b1r):
    def body(d_ref, x_ref, a_ref, b_ref, o_ref):
        d = lax.rsqrt(d_ref[...] + 1.0)
        o_ref[...] = jnp.maximum(
            d * a_ref[...] + d * d * x_ref[...] + b_ref[...], 0.0)

    sp = _ew_specs(3)
    sp["in_specs"].append(pl.BlockSpec((1, 1, P), lambda t, i: (0, 0, 0)))
    return pl.pallas_call(
        body, **sp,
        out_shape=jax.ShapeDtypeStruct((T, N, P), jnp.float32))(D1, xw, A1, b1r)


def _tc_score(AG, h1, Wrel_r, Wroot_r, brel_r):
    def body(ag_ref, h_ref, wr_ref, wo_ref, br_ref, o_ref):
        s = (jnp.sum(ag_ref[...] * wr_ref[...], axis=2, keepdims=True)
             + jnp.sum(h_ref[...] * wo_ref[...], axis=2, keepdims=True)
             + br_ref[...])
        o_ref[...] = jnp.broadcast_to(s, (1, BLKN, P))

    sp = _ew_specs(2)
    sp["in_specs"] += [pl.BlockSpec((1, 1, P), lambda t, i: (0, 0, 0)),
                       pl.BlockSpec((1, 1, P), lambda t, i: (0, 0, 0)),
                       pl.BlockSpec((1, 1, 1), lambda t, i: (0, 0, 0))]
    return pl.pallas_call(
        body, **sp,
        out_shape=jax.ShapeDtypeStruct((T, N, P), jnp.float32))(
            AG, h1, Wrel_r, Wroot_r, brel_r)


def _tc_keep(sp_row, sp_col):
    def body(r_ref, c_ref, o_ref):
        srow = r_ref[0]
        scol = c_ref[0]
        ii = lax.broadcasted_iota(jnp.int32, (NPAD, NPAD), 0)
        jj = lax.broadcasted_iota(jnp.int32, (NPAD, NPAD), 1)
        gt = (srow > scol).astype(jnp.float32)
        eqb = ((srow == scol) & (jj < ii)).astype(jnp.float32)
        cnt = jnp.sum(gt + eqb, axis=1, keepdims=True)
        keep = (cnt < float(KTOP)).astype(jnp.float32)
        o_ref[...] = jnp.broadcast_to(keep, (NPAD, P))[None]

    return pl.pallas_call(
        body,
        grid=(T * B,),
        in_specs=[pl.BlockSpec((1, 1, NPAD), lambda g: (g, 0, 0)),
                  pl.BlockSpec((1, NPAD, 1), lambda g: (g, 0, 0))],
        out_specs=pl.BlockSpec((1, NPAD, P), lambda g: (g, 0, 0)),
        out_shape=jax.ShapeDtypeStruct((T * B, NPAD, P), jnp.float32),
    )(sp_row, sp_col)


def _tc_y2xk(D2, keepw, h1, scw):
    def body(d_ref, k_ref, h_ref, s_ref, y_ref, xk_ref, dv_ref):
        d2 = k_ref[...] * lax.rsqrt(d_ref[...] + 1.0)
        xk = k_ref[...] * h_ref[...] * jnp.tanh(s_ref[...])
        xk_ref[...] = xk
        dv_ref[...] = d2
        y_ref[...] = d2 * xk

    sh = jax.ShapeDtypeStruct((T, N, P), jnp.float32)
    sp = _ew_specs(4)
    sp["out_specs"] = (sp["out_specs"],) * 3
    return pl.pallas_call(
        body, **sp, out_shape=(sh, sh, sh))(D2, keepw, h1, scw)


def _tc_emb(d2w, xk, A2, keepw, W2, b2r, sel):
    def body(d_ref, xk_ref, a_ref, k_ref, w_ref, b_ref, s_ref, o_ref):
        d2 = d_ref[0]
        pre = d2 * a_ref[0] + d2 * d2 * xk_ref[0]
        g = jnp.maximum(
            lax.dot_general(pre, w_ref[...], (((1,), (0,)), ((), ())),
                            preferred_element_type=jnp.float32,
                            precision=_HI) + b_ref[...], 0.0)
        g = g * k_ref[0][:, 0:1]
        o_ref[...] = lax.dot_general(
            s_ref[...], g, (((1,), (0,)), ((), ())),
            preferred_element_type=jnp.float32, precision=_HI)[None]

    return pl.pallas_call(
        body,
        grid=(T,),
        in_specs=[pl.BlockSpec((1, N, P), lambda t: (t, 0, 0)),
                  pl.BlockSpec((1, N, P), lambda t: (t, 0, 0)),
                  pl.BlockSpec((1, N, P), lambda t: (t, 0, 0)),
                  pl.BlockSpec((1, N, P), lambda t: (t, 0, 0)),
                  pl.BlockSpec((P, H), lambda t: (0, 0)),
                  pl.BlockSpec((1, H), lambda t: (0, 0)),
                  pl.BlockSpec((B, N), lambda t: (0, 0))],
        out_specs=pl.BlockSpec((1, B, H), lambda t: (t, 0, 0)),
        out_shape=jax.ShapeDtypeStruct((T, B, H), jnp.float32),
    )(d2w, xk, A2, keepw, W2, b2r, sel)


def _tc_lstm(emb, Wih, Whh, bihr, bhhr, Wo, bor):
    def body(s_ref, wi_ref, wh_ref, bi_ref, bh_ref, wo_ref, bo_ref, o_ref):
        h = jnp.zeros((B, H), jnp.float32)
        c = jnp.zeros((B, H), jnp.float32)
        for t in range(T):
            st = s_ref[t]
            gates = (lax.dot_general(st, wi_ref[...], (((1,), (1,)), ((), ())),
                                     preferred_element_type=jnp.float32,
                                     precision=_HI)
                     + lax.dot_general(h, wh_ref[...], (((1,), (1,)), ((), ())),
                                       preferred_element_type=jnp.float32,
                                       precision=_HI)
                     + bi_ref[...] + bh_ref[...])
            ig = jax.nn.sigmoid(gates[:, 0:H])
            fg = jax.nn.sigmoid(gates[:, H:2 * H])
            gg = jnp.tanh(gates[:, 2 * H:3 * H])
            og = jax.nn.sigmoid(gates[:, 3 * H:4 * H])
            c = fg * c + ig * gg
            h = og * jnp.tanh(c)
        o_ref[...] = jax.nn.sigmoid(
            lax.dot_general(h, wo_ref[...], (((1,), (0,)), ((), ())),
                            preferred_element_type=jnp.float32,
                            precision=_HI) + bo_ref[...])

    return pl.pallas_call(
        body,
        in_specs=[pl.BlockSpec((T, B, H), lambda: (0, 0, 0)),
                  pl.BlockSpec((4 * H, H), lambda: (0, 0)),
                  pl.BlockSpec((4 * H, H), lambda: (0, 0)),
                  pl.BlockSpec((1, 4 * H), lambda: (0, 0)),
                  pl.BlockSpec((1, 4 * H), lambda: (0, 0)),
                  pl.BlockSpec((H, 1), lambda: (0, 0)),
                  pl.BlockSpec((1, 1), lambda: (0, 0))],
        out_specs=pl.BlockSpec((B, 1), lambda: (0, 0)),
        out_shape=jax.ShapeDtypeStruct((B, 1), jnp.float32),
    )(emb, Wih, Whh, bihr, bhhr, Wo, bor)


def kernel(x, edge_index, edge_weight, W1, b1, Wrel, brel, Wroot, W2, b2,
           Wih, Whh, bih, bhh, Wo, bo):
    src = edge_index[:, 0, :].astype(jnp.int32)
    dst = edge_index[:, 1, :].astype(jnp.int32)
    padi = jnp.zeros((T, EP - E), jnp.int32)
    padf = jnp.zeros((T, EP - E), jnp.float32)
    src3 = jnp.concatenate([src, padi], axis=1).reshape(T, ROWS, LANES)
    dst3 = jnp.concatenate([dst, padi], axis=1).reshape(T, ROWS, LANES)
    w3 = jnp.concatenate([edge_weight, padf], axis=1).reshape(T, ROWS, LANES)
    zeros = jnp.zeros((NPSA, P), jnp.float32)
    sel = jnp.repeat(jnp.eye(B, dtype=jnp.float32), NPART, axis=1) / KTOP

    xw = _tc_xw(x, W1)
    D1 = _sc_splat(dst3, w3, zeros)
    y1 = _tc_y1(D1, xw)
    A1 = _sc_gs(src3, dst3, w3, y1, zeros)
    h1 = _tc_h1(D1, xw, A1, b1.reshape(1, 1, P))
    AG = _sc_gs(src3, dst3, w3, h1, zeros)
    scw = _tc_score(AG, h1, Wrel.reshape(1, 1, P), Wroot.reshape(1, 1, P),
                    brel.reshape(1, 1, 1))
    sc_nodes = scw[:, :, 0].reshape(T * B, NPART)
    sp = jnp.pad(sc_nodes, ((0, 0), (0, NPAD - NPART)), constant_values=-1e30)
    keep64 = _tc_keep(sp.reshape(T * B, 1, NPAD), sp.reshape(T * B, NPAD, 1))
    keepw = keep64[:, :NPART, :].reshape(T, N, P)
    D2 = _sc_gs(src3, dst3, w3, keepw, zeros)
    y2, xk, d2w = _tc_y2xk(D2, keepw, h1, scw)
    A2 = _sc_gs(src3, dst3, w3, y2, zeros)
    emb = _tc_emb(d2w, xk, A2, keepw, W2, b2.reshape(1, H), sel)
    return _tc_lstm(emb, Wih, Whh, bih.reshape(1, 4 * H), bhh.reshape(1, 4 * H),
                    Wo, bo.reshape(1, 1))


# speedup vs baseline: 1.5275x; 1.0611x over previous
import functools

import jax
import jax.numpy as jnp
from jax import lax
from jax.experimental import pallas as pl
from jax.experimental.pallas import tpu as pltpu
from jax.experimental.pallas import tpu_sc as plsc

T, N, F, P, H, B = 8, 10000, 128, 16, 128, 8
E = 320000
NPART = N // B
KTOP = 1000
NPAD = 1280

NSUB = 16
NCORE = 2
LANES = 128
EP = 327680
ROWS = EP // LANES
RPS = ROWS // NSUB
CH = 16
NCHUNK = RPS // CH
NPSA = 624
TAIL = N - NSUB * NPSA
TPC = T // NCORE

@functools.cache
def _sc_params():
    import dataclasses
    cp = pltpu.CompilerParams(use_tc_tiling_on_sc=False)
    if "needs_layout_passes" in pltpu.CompilerParams.__dataclass_fields__:
        cp = dataclasses.replace(cp, needs_layout_passes=False)
    return cp


@functools.cache
def _mesh():
    return plsc.VectorSubcoreMesh(core_axis_name="core",
                                  subcore_axis_name="subcore",
                                  num_cores=NCORE, num_subcores=NSUB)
_HI = jax.lax.Precision.HIGHEST


def _sc_scratch():
    buf = []
    for _ in range(2):
        buf += [
            pltpu.VMEM((CH, LANES), jnp.int32),
            pltpu.VMEM((CH, LANES), jnp.int32),
            pltpu.VMEM((CH, LANES), jnp.float32),
            pltpu.VMEM((CH * LANES, P), jnp.float32),
            pltpu.SemaphoreType.DMA,
            pltpu.SemaphoreType.DMA,
            pltpu.SemaphoreType.DMA,
        ]
    return [pltpu.VMEM_SHARED((N, P), jnp.float32)] + buf


def _scale_rows(gather, rows, wbuf):
    @pl.loop(0, CH)
    def _mr(r):
        @pl.loop(0, LANES // P)
        def _mg(g):
            w16 = wbuf[r, pl.ds(g * P, P)]

            @pl.loop(0, P, unroll=8)
            def _ml(l):
                lsp = jnp.zeros((P, 1), jnp.int32) + l
                wsp = lax.gather(
                    w16, lsp,
                    lax.GatherDimensionNumbers(
                        offset_dims=(), collapsed_slice_dims=(0,),
                        start_index_map=(0,)),
                    (1,), mode=lax.GatherScatterMode.PROMISE_IN_BOUNDS)
                e = r * LANES + g * P + l
                if gather:
                    rows[e] = rows[e] * wsp
                else:
                    rows[e] = wsp


def _sc_body(gather, t, src_h, dst_h, w_h, y_h, z_h, out_h,
             accum, b0, b1, sid, nbase):
    pltpu.sync_copy(z_h, accum.at[pl.ds(nbase, NPSA)])

    @pl.when(sid == NSUB - 1)
    def _zt():
        pltpu.sync_copy(z_h.at[pl.ds(0, TAIL)],
                        accum.at[pl.ds(NSUB * NPSA, TAIL)])
    plsc.subcore_barrier()

    def fire_lin(j, bufs):
        sidx, didx, wbuf, rows, lsem, gsem, ssem = bufs
        rb = sid * RPS + j * CH
        hs = [pltpu.async_copy(dst_h.at[t, pl.ds(rb, CH)], didx, lsem),
              pltpu.async_copy(w_h.at[t, pl.ds(rb, CH)], wbuf, lsem)]
        if gather:
            hs.append(pltpu.async_copy(src_h.at[t, pl.ds(rb, CH)], sidx, lsem))
        return hs

    def fire_gather(bufs):
        sidx, didx, wbuf, rows, lsem, gsem, ssem = bufs
        if not gather:
            return []
        return [pltpu.async_copy(y_h.at[t].at[sidx.at[r]],
                                 rows.at[pl.ds(r * LANES, LANES)], gsem)
                for r in range(CH)]

    def fire_scatter(bufs):
        sidx, didx, wbuf, rows, lsem, gsem, ssem = bufs
        return [pltpu.async_copy(rows.at[pl.ds(r * LANES, LANES)],
                                 accum.at[didx.at[r]], ssem, add=True)
                for r in range(CH)]

    @pl.loop(0, NCHUNK // 2)
    def _c(k):
        a, b = 2 * k, 2 * k + 1
        la = fire_lin(a, b0)
        lb = fire_lin(b, b1)
        for h in la:
            h.wait()
        ga = fire_gather(b0)
        for h in lb:
            h.wait()
        for h in ga:
            h.wait()
        gb = fire_gather(b1)
        _scale_rows(gather, b0[3], b0[2])
        sa = fire_scatter(b0)
        for h in gb:
            h.wait()
        _scale_rows(gather, b1[3], b1[2])
        for h in sa:
            h.wait()
        sb = fire_scatter(b1)
        for h in sb:
            h.wait()

    plsc.subcore_barrier()
    pltpu.sync_copy(accum.at[pl.ds(nbase, NPSA)],
                    out_h.at[t, pl.ds(nbase, NPSA)])

    @pl.when(sid == NSUB - 1)
    def _ot():
        pltpu.sync_copy(accum.at[pl.ds(NSUB * NPSA, TAIL)],
                        out_h.at[t, pl.ds(NSUB * NPSA, TAIL)])


def _sc_gs(src3, dst3, w3, y, zeros):
    @functools.partial(
        pl.kernel,
        out_type=jax.ShapeDtypeStruct((T, N, P), jnp.float32),
        mesh=_mesh(),
        scratch_types=_sc_scratch(),
        compiler_params=_sc_params(),
    )
    def k(src_h, dst_h, w_h, y_h, z_h, out_h, accum, *bufs):
        cid = lax.axis_index("core")
        sid = lax.axis_index("subcore")
        nbase = sid * NPSA

        @pl.loop(0, TPC)
        def _t(i):
            t = cid * TPC + i
            _sc_body(True, t, src_h, dst_h, w_h, y_h, z_h, out_h,
                     accum, bufs[:7], bufs[7:], sid, nbase)

    return k(src3, dst3, w3, y, zeros)


def _sc_splat(dst3, w3, zeros):
    @functools.partial(
        pl.kernel,
        out_type=jax.ShapeDtypeStruct((T, N, P), jnp.float32),
        mesh=_mesh(),
        scratch_types=_sc_scratch(),
        compiler_params=_sc_params(),
    )
    def k(dst_h, w_h, z_h, out_h, accum, *bufs):
        cid = lax.axis_index("core")
        sid = lax.axis_index("subcore")
        nbase = sid * NPSA

        @pl.loop(0, TPC)
        def _t(i):
            t = cid * TPC + i
            _sc_body(False, t, dst_h, dst_h, w_h, None, z_h, out_h,
                     accum, bufs[:7], bufs[7:], sid, nbase)

    return k(dst3, w3, zeros)


BLKN = 2000


def _tc_xw(x, W1):
    def body(x_ref, w_ref, o_ref):
        o_ref[...] = lax.dot_general(
            x_ref[0], w_ref[...], (((1,), (0,)), ((), ())),
            preferred_element_type=jnp.float32, precision=_HI)[None]

    return pl.pallas_call(
        body,
        grid=(T, N // BLKN),
        in_specs=[pl.BlockSpec((1, BLKN, F), lambda t, i: (t, i, 0)),
                  pl.BlockSpec((F, P), lambda t, i: (0, 0))],
        out_specs=pl.BlockSpec((1, BLKN, P), lambda t, i: (t, i, 0)),
        out_shape=jax.ShapeDtypeStruct((T, N, P), jnp.float32),
    )(x, W1)


def _ew_specs(n_in):
    return dict(
        grid=(T, N // BLKN),
        in_specs=[pl.BlockSpec((1, BLKN, P), lambda t, i: (t, i, 0))
                  for _ in range(n_in)],
        out_specs=pl.BlockSpec((1, BLKN, P), lambda t, i: (t, i, 0)),
    )


def _tc_y1(D1, xw):
    def body(d_ref, x_ref, o_ref):
        d = lax.rsqrt(d_ref[...] + 1.0)
        o_ref[...] = d * x_ref[...]

    return pl.pallas_call(
        body, **_ew_specs(2),
        out_shape=jax.ShapeDtypeStruct((T, N, P), jnp.float32))(D1, xw)


def _tc_h1(D1, xw, A1, b1r):
    def body(d_ref, x_ref, a_ref, b_ref, o_ref):
        d = lax.rsqrt(d_ref[...] + 1.0)
        o_ref[...] = jnp.maximum(
            d * a_ref[...] + d * d * x_ref[...] + b_ref[...], 0.0)

    sp = _ew_specs(3)
    sp["in_specs"].append(pl.BlockSpec((1, 1, P), lambda t, i: (0, 0, 0)))
    return pl.pallas_call(
        body, **sp,
        out_shape=jax.ShapeDtypeStruct((T, N, P), jnp.float32))(D1, xw, A1, b1r)


def _tc_score(AG, h1, Wrel_r, Wroot_r, brel_r):
    def body(ag_ref, h_ref, wr_ref, wo_ref, br_ref, o_ref):
        s = (jnp.sum(ag_ref[...] * wr_ref[...], axis=2, keepdims=True)
             + jnp.sum(h_ref[...] * wo_ref[...], axis=2, keepdims=True)
             + br_ref[...])
        o_ref[...] = jnp.broadcast_to(s, (1, BLKN, P))

    sp = _ew_specs(2)
    sp["in_specs"] += [pl.BlockSpec((1, 1, P), lambda t, i: (0, 0, 0)),
                       pl.BlockSpec((1, 1, P), lambda t, i: (0, 0, 0)),
                       pl.BlockSpec((1, 1, 1), lambda t, i: (0, 0, 0))]
    return pl.pallas_call(
        body, **sp,
        out_shape=jax.ShapeDtypeStruct((T, N, P), jnp.float32))(
            AG, h1, Wrel_r, Wroot_r, brel_r)


def _tc_keep(sp_row, sp_col):
    def body(r_ref, c_ref, o_ref):
        srow = r_ref[0]
        scol = c_ref[0]
        ii = lax.broadcasted_iota(jnp.int32, (NPAD, NPAD), 0)
        jj = lax.broadcasted_iota(jnp.int32, (NPAD, NPAD), 1)
        gt = (srow > scol).astype(jnp.float32)
        eqb = ((srow == scol) & (jj < ii)).astype(jnp.float32)
        cnt = jnp.sum(gt + eqb, axis=1, keepdims=True)
        keep = (cnt < float(KTOP)).astype(jnp.float32)
        o_ref[...] = jnp.broadcast_to(keep, (NPAD, P))[None]

    return pl.pallas_call(
        body,
        grid=(T * B,),
        in_specs=[pl.BlockSpec((1, 1, NPAD), lambda g: (g, 0, 0)),
                  pl.BlockSpec((1, NPAD, 1), lambda g: (g, 0, 0))],
        out_specs=pl.BlockSpec((1, NPAD, P), lambda g: (g, 0, 0)),
        out_shape=jax.ShapeDtypeStruct((T * B, NPAD, P), jnp.float32),
    )(sp_row, sp_col)


def _tc_y2xk(D2, keepw, h1, scw):
    def body(d_ref, k_ref, h_ref, s_ref, y_ref, xk_ref, dv_ref):
        d2 = k_ref[...] * lax.rsqrt(d_ref[...] + 1.0)
        xk = k_ref[...] * h_ref[...] * jnp.tanh(s_ref[...])
        xk_ref[...] = xk
        dv_ref[...] = d2
        y_ref[...] = d2 * xk

    sh = jax.ShapeDtypeStruct((T, N, P), jnp.float32)
    sp = _ew_specs(4)
    sp["out_specs"] = (sp["out_specs"],) * 3
    return pl.pallas_call(
        body, **sp, out_shape=(sh, sh, sh))(D2, keepw, h1, scw)


def _tc_emb(d2w, xk, A2, keepw, W2, b2r, sel):
    def body(d_ref, xk_ref, a_ref, k_ref, w_ref, b_ref, s_ref, o_ref):
        d2 = d_ref[0]
        pre = d2 * a_ref[0] + d2 * d2 * xk_ref[0]
        g = jnp.maximum(
            lax.dot_general(pre, w_ref[...], (((1,), (0,)), ((), ())),
                            preferred_element_type=jnp.float32,
                            precision=_HI) + b_ref[...], 0.0)
        g = g * k_ref[0][:, 0:1]
        o_ref[...] = lax.dot_general(
            s_ref[...], g, (((1,), (0,)), ((), ())),
            preferred_element_type=jnp.float32, precision=_HI)[None]

    return pl.pallas_call(
        body,
        grid=(T,),
        in_specs=[pl.BlockSpec((1, N, P), lambda t: (t, 0, 0)),
                  pl.BlockSpec((1, N, P), lambda t: (t, 0, 0)),
                  pl.BlockSpec((1, N, P), lambda t: (t, 0, 0)),
                  pl.BlockSpec((1, N, P), lambda t: (t, 0, 0)),
                  pl.BlockSpec((P, H), lambda t: (0, 0)),
                  pl.BlockSpec((1, H), lambda t: (0, 0)),
                  pl.BlockSpec((B, N), lambda t: (0, 0))],
        out_specs=pl.BlockSpec((1, B, H), lambda t: (t, 0, 0)),
        out_shape=jax.ShapeDtypeStruct((T, B, H), jnp.float32),
    )(d2w, xk, A2, keepw, W2, b2r, sel)


def _tc_lstm(emb, Wih, Whh, bihr, bhhr, Wo, bor):
    def body(s_ref, wi_ref, wh_ref, bi_ref, bh_ref, wo_ref, bo_ref, o_ref):
        h = jnp.zeros((B, H), jnp.float32)
        c = jnp.zeros((B, H), jnp.float32)
        for t in range(T):
            st = s_ref[t]
            gates = (lax.dot_general(st, wi_ref[...], (((1,), (1,)), ((), ())),
                                     preferred_element_type=jnp.float32,
                                     precision=_HI)
                     + lax.dot_general(h, wh_ref[...], (((1,), (1,)), ((), ())),
                                       preferred_element_type=jnp.float32,
                                       precision=_HI)
                     + bi_ref[...] + bh_ref[...])
            ig = jax.nn.sigmoid(gates[:, 0:H])
            fg = jax.nn.sigmoid(gates[:, H:2 * H])
            gg = jnp.tanh(gates[:, 2 * H:3 * H])
            og = jax.nn.sigmoid(gates[:, 3 * H:4 * H])
            c = fg * c + ig * gg
            h = og * jnp.tanh(c)
        o_ref[...] = jax.nn.sigmoid(
            lax.dot_general(h, wo_ref[...], (((1,), (0,)), ((), ())),
                            preferred_element_type=jnp.float32,
                            precision=_HI) + bo_ref[...])

    return pl.pallas_call(
        body,
        in_specs=[pl.BlockSpec((T, B, H), lambda: (0, 0, 0)),
                  pl.BlockSpec((4 * H, H), lambda: (0, 0)),
                  pl.BlockSpec((4 * H, H), lambda: (0, 0)),
                  pl.BlockSpec((1, 4 * H), lambda: (0, 0)),
                  pl.BlockSpec((1, 4 * H), lambda: (0, 0)),
                  pl.BlockSpec((H, 1), lambda: (0, 0)),
                  pl.BlockSpec((1, 1), lambda: (0, 0))],
        out_specs=pl.BlockSpec((B, 1), lambda: (0, 0)),
        out_shape=jax.ShapeDtypeStruct((B, 1), jnp.float32),
    )(emb, Wih, Whh, bihr, bhhr, Wo, bor)


def kernel(x, edge_index, edge_weight, W1, b1, Wrel, brel, Wroot, W2, b2,
           Wih, Whh, bih, bhh, Wo, bo):
    src = edge_index[:, 0, :].astype(jnp.int32)
    dst = edge_index[:, 1, :].astype(jnp.int32)
    padi = jnp.zeros((T, EP - E), jnp.int32)
    padf = jnp.zeros((T, EP - E), jnp.float32)
    src3 = jnp.concatenate([src, padi], axis=1).reshape(T, ROWS, LANES)
    dst3 = jnp.concatenate([dst, padi], axis=1).reshape(T, ROWS, LANES)
    w3 = jnp.concatenate([edge_weight, padf], axis=1).reshape(T, ROWS, LANES)
    zeros = jnp.zeros((NPSA, P), jnp.float32)
    sel = jnp.repeat(jnp.eye(B, dtype=jnp.float32), NPART, axis=1) / KTOP

    xw = _tc_xw(x, W1)
    D1 = _sc_splat(dst3, w3, zeros)
    y1 = _tc_y1(D1, xw)
    A1 = _sc_gs(src3, dst3, w3, y1, zeros)
    h1 = _tc_h1(D1, xw, A1, b1.reshape(1, 1, P))
    AG = _sc_gs(src3, dst3, w3, h1, zeros)
    scw = _tc_score(AG, h1, Wrel.reshape(1, 1, P), Wroot.reshape(1, 1, P),
                    brel.reshape(1, 1, 1))
    sc_nodes = scw[:, :, 0].reshape(T * B, NPART)
    sp = jnp.pad(sc_nodes, ((0, 0), (0, NPAD - NPART)), constant_values=-1e30)
    keep64 = _tc_keep(sp.reshape(T * B, 1, NPAD), sp.reshape(T * B, NPAD, 1))
    keepw = keep64[:, :NPART, :].reshape(T, N, P)
    D2 = _sc_gs(src3, dst3, w3, keepw, zeros)
    y2, xk, d2w = _tc_y2xk(D2, keepw, h1, scw)
    A2 = _sc_gs(src3, dst3, w3, y2, zeros)
    emb = _tc_emb(d2w, xk, A2, keepw, W2, b2.reshape(1, H), sel)
    return _tc_lstm(emb, Wih, Whh, bih.reshape(1, 4 * H), bhh.reshape(1, 4 * H),
                    Wo, bo.reshape(1, 1))
